# baseline (device time: 111992 ns/iter reference)
import jax
import jax.numpy as jnp
from jax import lax
from jax.experimental import pallas as pl
from jax.experimental.pallas import tpu as pltpu

N_DEV = 4
S_LOC = 512
S_GLOB = N_DEV * S_LOC
SB = 256
D = 1024
HL = 8
DH = 128
SCALE = 0.08838834764831843
NSLOT = 2 * (N_DEV - 1)

BF16 = jnp.bfloat16
F32 = jnp.float32
MESHID = pl.DeviceIdType.MESH


def kernel(x, Wq, Wo, Wk, Wv):
    x2 = x.reshape(S_LOC, D).astype(BF16)
    wq = Wq.astype(BF16)
    wk = Wk.astype(BF16)
    wv = Wv.astype(BF16)
    wo = Wo.astype(BF16)

    def body(x_ref, wq_ref, wo_ref, wk_ref, wv_ref, out_ref,
             xg, kf, vf, qfr, obr, sbuf, rbuf,
             ag_send, ag_recv, rs_send, rs_recv):
        my = lax.axis_index("i")

        barrier = pltpu.get_barrier_semaphore()
        for d in (1, 2, 3):
            pl.semaphore_signal(
                barrier, inc=1,
                device_id=(lax.rem(my + d, N_DEV),), device_id_type=MESHID,
            )
        pl.semaphore_wait(barrier, 3)

        my_rows = pl.ds(my * S_LOC, S_LOC)
        xg[my_rows, :] = x_ref[...]
        sends = []
        for d in (1, 2, 3):
            rd = pltpu.make_async_remote_copy(
                src_ref=xg.at[my_rows, :],
                dst_ref=xg.at[my_rows, :],
                send_sem=ag_send.at[d - 1],
                recv_sem=ag_recv.at[d - 1],
                device_id=(lax.rem(my + d, N_DEV),),
                device_id_type=MESHID,
            )
            rd.start()
            sends.append(rd)

        kf[my_rows, :] = jnp.dot(xg[my_rows, :], wk_ref[...],
                                 preferred_element_type=F32).astype(BF16)
        vf[my_rows, :] = jnp.dot(xg[my_rows, :], wv_ref[...],
                                 preferred_element_type=F32).astype(BF16)

        for d in (1, 3, 2):
            o_rows = pl.ds(lax.rem(my + N_DEV - d, N_DEV) * S_LOC, S_LOC)
            rcv = pltpu.make_async_remote_copy(
                src_ref=xg.at[o_rows, :],
                dst_ref=xg.at[o_rows, :],
                send_sem=ag_send.at[d - 1],
                recv_sem=ag_recv.at[d - 1],
                device_id=(my,), device_id_type=MESHID,
            )
            rcv.wait_recv()
            kf[o_rows, :] = jnp.dot(xg[o_rows, :], wk_ref[...],
                                    preferred_element_type=F32).astype(BF16)
            vf[o_rows, :] = jnp.dot(xg[o_rows, :], wv_ref[...],
                                    preferred_element_type=F32).astype(BF16)

        def attn_block(rows):
            qfr[...] = (jnp.dot(xg[rows, :], wq_ref[...],
                                preferred_element_type=F32)
                        * SCALE).astype(BF16)

            def hbody(h, _):
                hs = pl.ds(h * DH, DH)
                s = lax.dot_general(
                    qfr[:, hs], kf[:, hs], (((1,), (1,)), ((), ())),
                    preferred_element_type=F32,
                )
                p = jnp.exp(s)
                l = jnp.sum(p, axis=-1, keepdims=True)
                ob = lax.dot_general(
                    p.astype(BF16), vf[:, hs], (((1,), (0,)), ((), ())),
                    preferred_element_type=F32,
                )
                obr[:, hs] = (ob / l).astype(BF16)
                return 0

            lax.fori_loop(0, HL, hbody, 0)
            return jnp.dot(obr[...], wo_ref[...], preferred_element_type=F32)

        for d in (2, 1, 3):
            tgt = lax.rem(my + d, N_DEV)
            for s2 in (0, 1):
                slot = (d - 1) * 2 + s2
                sbuf[slot, :, :] = attn_block(
                    pl.ds(tgt * S_LOC + s2 * SB, SB)).astype(BF16)
                rd = pltpu.make_async_remote_copy(
                    src_ref=sbuf.at[slot],
                    dst_ref=rbuf.at[slot],
                    send_sem=rs_send.at[slot],
                    recv_sem=rs_recv.at[slot],
                    device_id=(tgt,), device_id_type=MESHID,
                )
                rd.start()
                sends.append(rd)

        own0 = attn_block(pl.ds(my * S_LOC, SB))
        own1 = attn_block(pl.ds(my * S_LOC + SB, SB))

        for slot in range(NSLOT):
            rcv = pltpu.make_async_remote_copy(
                src_ref=sbuf.at[slot],
                dst_ref=rbuf.at[slot],
                send_sem=rs_send.at[slot],
                recv_sem=rs_recv.at[slot],
                device_id=(my,), device_id_type=MESHID,
            )
            rcv.wait_recv()
        out_ref[pl.ds(0, SB), :] = (own0
                                    + rbuf[0].astype(F32)
                                    + rbuf[2].astype(F32)
                                    + rbuf[4].astype(F32))
        out_ref[pl.ds(SB, SB), :] = (own1
                                     + rbuf[1].astype(F32)
                                     + rbuf[3].astype(F32)
                                     + rbuf[5].astype(F32))

        for rd in sends:
            rd.wait_send()

    out = pl.pallas_call(
        body,
        out_shape=jax.ShapeDtypeStruct((S_LOC, D), F32),
        in_specs=[pl.BlockSpec(memory_space=pltpu.VMEM)] * 5,
        out_specs=pl.BlockSpec(memory_space=pltpu.VMEM),
        scratch_shapes=[
            pltpu.VMEM((S_GLOB, D), BF16),
            pltpu.VMEM((S_GLOB, D), BF16),
            pltpu.VMEM((S_GLOB, D), BF16),
            pltpu.VMEM((SB, D), BF16),
            pltpu.VMEM((SB, D), BF16),
            pltpu.VMEM((NSLOT, SB, D), BF16),
            pltpu.VMEM((NSLOT, SB, D), BF16),
            pltpu.SemaphoreType.DMA((N_DEV - 1,)),
            pltpu.SemaphoreType.DMA((N_DEV - 1,)),
            pltpu.SemaphoreType.DMA((NSLOT,)),
            pltpu.SemaphoreType.DMA((NSLOT,)),
        ],
        compiler_params=pltpu.CompilerParams(collective_id=0),
    )(x2, wq, wo, wk, wv)
    return out.reshape(1, S_LOC, D)


# device time: 99966 ns/iter; 1.1203x vs baseline; 1.1203x over previous
import jax
import jax.numpy as jnp
from jax import lax
from jax.experimental import pallas as pl
from jax.experimental.pallas import tpu as pltpu

N_DEV = 4
S_LOC = 512
S_GLOB = N_DEV * S_LOC
SB = 512
D = 1024
HL = 8
DH = 128
SCALE = 0.08838834764831843
NSLOT = N_DEV - 1

BF16 = jnp.bfloat16
F32 = jnp.float32
MESHID = pl.DeviceIdType.MESH


def kernel(x, Wq, Wo, Wk, Wv):
    x2 = x.reshape(S_LOC, D).astype(BF16)
    wq = Wq.astype(BF16)
    wk = Wk.astype(BF16)
    wv = Wv.astype(BF16)
    wo = Wo.astype(BF16)

    def body(x_ref, wq_ref, wo_ref, wk_ref, wv_ref, out_ref,
             xg, kf, vf, qfr, obr, sbuf, rbuf,
             ag_send, ag_recv, rs_send, rs_recv):
        my = lax.axis_index("i")

        barrier = pltpu.get_barrier_semaphore()
        for d in (1, 2, 3):
            pl.semaphore_signal(
                barrier, inc=1,
                device_id=(lax.rem(my + d, N_DEV),), device_id_type=MESHID,
            )
        pl.semaphore_wait(barrier, 3)

        my_rows = pl.ds(my * S_LOC, S_LOC)
        xg[my_rows, :] = x_ref[...]
        sends = []
        for d in (1, 2, 3):
            rd = pltpu.make_async_remote_copy(
                src_ref=xg.at[my_rows, :],
                dst_ref=xg.at[my_rows, :],
                send_sem=ag_send.at[d - 1],
                recv_sem=ag_recv.at[d - 1],
                device_id=(lax.rem(my + d, N_DEV),),
                device_id_type=MESHID,
            )
            rd.start()
            sends.append(rd)

        kf[my_rows, :] = jnp.dot(xg[my_rows, :], wk_ref[...],
                                 preferred_element_type=F32).astype(BF16)
        vf[my_rows, :] = jnp.dot(xg[my_rows, :], wv_ref[...],
                                 preferred_element_type=F32).astype(BF16)

        for d in (1, 3, 2):
            o_rows = pl.ds(lax.rem(my + N_DEV - d, N_DEV) * S_LOC, S_LOC)
            rcv = pltpu.make_async_remote_copy(
                src_ref=xg.at[o_rows, :],
                dst_ref=xg.at[o_rows, :],
                send_sem=ag_send.at[d - 1],
                recv_sem=ag_recv.at[d - 1],
                device_id=(my,), device_id_type=MESHID,
            )
            rcv.wait_recv()
            kf[o_rows, :] = jnp.dot(xg[o_rows, :], wk_ref[...],
                                    preferred_element_type=F32).astype(BF16)
            vf[o_rows, :] = jnp.dot(xg[o_rows, :], wv_ref[...],
                                    preferred_element_type=F32).astype(BF16)

        def attn_block(rows):
            qfr[...] = (jnp.dot(xg[rows, :], wq_ref[...],
                                preferred_element_type=F32)
                        * SCALE).astype(BF16)

            def hbody(h, _):
                hs = pl.ds(h * DH, DH)
                s = lax.dot_general(
                    qfr[:, hs], kf[:, hs], (((1,), (1,)), ((), ())),
                    preferred_element_type=F32,
                )
                p = jnp.exp(s)
                l = jnp.sum(p, axis=-1, keepdims=True)
                ob = lax.dot_general(
                    p.astype(BF16), vf[:, hs], (((1,), (0,)), ((), ())),
                    preferred_element_type=F32,
                )
                obr[:, hs] = (ob / l).astype(BF16)
                return 0

            lax.fori_loop(0, HL, hbody, 0)
            return jnp.dot(obr[...], wo_ref[...], preferred_element_type=F32)

        for d in (2, 1, 3):
            tgt = lax.rem(my + d, N_DEV)
            slot = d - 1
            sbuf[slot, :, :] = attn_block(
                pl.ds(tgt * S_LOC, S_LOC)).astype(BF16)
            rd = pltpu.make_async_remote_copy(
                src_ref=sbuf.at[slot],
                dst_ref=rbuf.at[slot],
                send_sem=rs_send.at[slot],
                recv_sem=rs_recv.at[slot],
                device_id=(tgt,), device_id_type=MESHID,
            )
            rd.start()
            sends.append(rd)

        own = attn_block(my_rows)

        for slot in range(NSLOT):
            rcv = pltpu.make_async_remote_copy(
                src_ref=sbuf.at[slot],
                dst_ref=rbuf.at[slot],
                send_sem=rs_send.at[slot],
                recv_sem=rs_recv.at[slot],
                device_id=(my,), device_id_type=MESHID,
            )
            rcv.wait_recv()
        out_ref[...] = (own
                        + rbuf[0].astype(F32)
                        + rbuf[1].astype(F32)
                        + rbuf[2].astype(F32))

        for rd in sends:
            rd.wait_send()

    out = pl.pallas_call(
        body,
        out_shape=jax.ShapeDtypeStruct((S_LOC, D), F32),
        in_specs=[pl.BlockSpec(memory_space=pltpu.VMEM)] * 5,
        out_specs=pl.BlockSpec(memory_space=pltpu.VMEM),
        scratch_shapes=[
            pltpu.VMEM((S_GLOB, D), BF16),
            pltpu.VMEM((S_GLOB, D), BF16),
            pltpu.VMEM((S_GLOB, D), BF16),
            pltpu.VMEM((SB, D), BF16),
            pltpu.VMEM((SB, D), BF16),
            pltpu.VMEM((NSLOT, SB, D), BF16),
            pltpu.VMEM((NSLOT, SB, D), BF16),
            pltpu.SemaphoreType.DMA((N_DEV - 1,)),
            pltpu.SemaphoreType.DMA((N_DEV - 1,)),
            pltpu.SemaphoreType.DMA((NSLOT,)),
            pltpu.SemaphoreType.DMA((NSLOT,)),
        ],
        compiler_params=pltpu.CompilerParams(collective_id=0),
    )(x2, wq, wo, wk, wv)
    return out.reshape(1, S_LOC, D)
